# baseline (speedup 1.0000x reference)
import functools
import jax
import jax.numpy as jnp
from jax import lax
from jax.experimental import pallas as pl
from jax.experimental.pallas import tpu as pltpu
from jax.experimental.pallas import tpu_sc as plsc

N, P, D = 10000, 8, 256
L = 16
T = D // L
NC, NS = 2, 16
NW = NC * NS
CH = 16

SC_BASE = 5
SC_EXTRA = 20
SC_NCH = NW * SC_BASE + SC_EXTRA
S = SC_NCH * CH
A = N - S
HALF = (SC_BASE - 1) // 2
CH0 = A // CH

BN = 712
NB = A // BN
BN2 = 2000
NB2 = N // BN2


def _worker_start(wid):
    return CH0 + wid * SC_BASE + jnp.minimum(wid, SC_EXTRA)


def _sc_mesh():
    return plsc.VectorSubcoreMesh(core_axis_name="c", subcore_axis_name="s")


def _in_start(h_hbm, cid, buf, sem):
    pltpu.async_copy(h_hbm.at[pl.ds(cid * CH, CH)], buf, sem)


def _in_wait(h_hbm, buf, sem):
    pltpu.make_async_copy(h_hbm.at[pl.ds(0, CH)], buf, sem).wait()


def _p1sc_body(h_hbm, w_hbm, out_hbm, wbuf, buf0, buf1, stage, sem0, sem1):
    wid = lax.axis_index("s") * NC + lax.axis_index("c")
    pltpu.sync_copy(w_hbm, wbuf)
    wv = [wbuf[pl.ds(t * L, L)] for t in range(T)]
    start = _worker_start(wid)

    def rows(buf, acc):
        def row_body(r, acc):
            acc = list(acc)
            for p in range(P):
                a = acc[p]
                for t in range(T):
                    a = a + buf[r, p, pl.ds(t * L, L)] * wv[t]
                acc[p] = a
            return tuple(acc)
        return lax.fori_loop(0, CH, row_body, acc)

    _in_start(h_hbm, start, buf0, sem0)

    def body2(k, acc):
        g = 2 * k
        _in_start(h_hbm, start + g + 1, buf1, sem1)
        _in_wait(h_hbm, buf0, sem0)
        acc = rows(buf0, acc)
        _in_start(h_hbm, start + g + 2, buf0, sem0)
        _in_wait(h_hbm, buf1, sem1)
        return rows(buf1, acc)

    acc0 = tuple(jnp.zeros((L,), jnp.float32) for _ in range(P))
    acc = lax.fori_loop(0, HALF, body2, acc0)
    _in_wait(h_hbm, buf0, sem0)
    acc = rows(buf0, acc)
    for p in range(P):
        stage[p, :] = acc[p]

    @pl.when(wid < SC_EXTRA)
    def _extra():
        pltpu.sync_copy(h_hbm.at[pl.ds((start + SC_BASE) * CH, CH)], buf1)
        acc_e = rows(buf1, acc0)
        for p in range(P):
            stage[p, :] = stage[p, :] + acc_e[p]

    pltpu.sync_copy(stage, out_hbm.at[wid])


def _p1sc(h, Wf):
    f = pl.kernel(
        _p1sc_body,
        out_type=jax.ShapeDtypeStruct((NW, P, L), jnp.float32),
        mesh=_sc_mesh(),
        scratch_types=[
            pltpu.VMEM((D,), jnp.float32),
            pltpu.VMEM((CH, P, D), jnp.float32),
            pltpu.VMEM((CH, P, D), jnp.float32),
            pltpu.VMEM((P, L), jnp.float32),
            pltpu.SemaphoreType.DMA,
            pltpu.SemaphoreType.DMA,
        ],
    )
    return f(h, Wf)


def _p1tc_body(h_ref, zphi_ref, acc_ref):
    i = pl.program_id(0)

    @pl.when(i == 0)
    def _init():
        acc_ref[...] = jnp.zeros_like(acc_ref)

    acc_ref[...] += jnp.sum(h_ref[...], axis=0)

    @pl.when(i == NB - 1)
    def _fin():
        zphi_ref[...] = acc_ref[...]


def _p1tc(h):
    return pl.pallas_call(
        _p1tc_body,
        grid=(NB,),
        in_specs=[pl.BlockSpec((BN, P, D), lambda i: (i, 0, 0))],
        out_specs=pl.BlockSpec((P, D), lambda i: (0, 0)),
        out_shape=jax.ShapeDtypeStruct((P, D), jnp.float32),
        scratch_shapes=[pltpu.VMEM((P, D), jnp.float32)],
    )(h)


def _p2tc_body(zphi_ref, w_ref, wpart_ref, h_ref, z_ref, beta_ref):
    i = pl.program_id(0)

    @pl.when(i == 0)
    def _betastep():
        w = jnp.dot(zphi_ref[...], w_ref[...])
        w = w + jnp.sum(wpart_ref[...], axis=(0, 2)).reshape(P, 1)
        w = jnp.where(w >= 0, w, 0.01 * w)
        m = jnp.max(w, axis=0, keepdims=True)
        e = jnp.exp(w - m)
        beta = e / jnp.sum(e, axis=0, keepdims=True)
        beta_ref[...] = jnp.broadcast_to(beta, (P, D))

    z_ref[...] = jnp.sum(h_ref[...] * beta_ref[...][None, :, :], axis=1)


def _p2tc(zphi, Wm, wpart, h):
    return pl.pallas_call(
        _p2tc_body,
        grid=(NB2,),
        in_specs=[
            pl.BlockSpec((P, D), lambda i: (0, 0)),
            pl.BlockSpec((D, 1), lambda i: (0, 0)),
            pl.BlockSpec((NW, P, L), lambda i: (0, 0, 0)),
            pl.BlockSpec((BN2, P, D), lambda i: (i, 0, 0)),
        ],
        out_specs=pl.BlockSpec((BN2, D), lambda i: (i, 0)),
        out_shape=jax.ShapeDtypeStruct((N, D), jnp.float32),
        scratch_shapes=[pltpu.VMEM((P, D), jnp.float32)],
    )(zphi, Wm, wpart, h)


def kernel(h, W):
    wpart = _p1sc(h, W.reshape(D))
    zphi = _p1tc(h)
    return _p2tc(zphi, W, wpart, h)


# speedup vs baseline: 1.0246x; 1.0246x over previous
import functools
import jax
import jax.numpy as jnp
from jax import lax
from jax.experimental import pallas as pl
from jax.experimental.pallas import tpu as pltpu
from jax.experimental.pallas import tpu_sc as plsc

N, P, D = 10000, 8, 256
L = 16
T = D // L
NC, NS = 2, 16
NW = NC * NS
CH = 16

SC_BASE = 5
SC_EXTRA = 10
SC_NCH = NW * SC_BASE + SC_EXTRA
S = SC_NCH * CH
A = N - S
HALF = (SC_BASE - 1) // 2
CH0 = A // CH

BN = 728
NB = A // BN
BN2 = 2000
NB2 = N // BN2


def _worker_start(wid):
    return CH0 + wid * SC_BASE + jnp.minimum(wid, SC_EXTRA)


def _sc_mesh():
    return plsc.VectorSubcoreMesh(core_axis_name="c", subcore_axis_name="s")


def _in_start(h_hbm, cid, buf, sem):
    pltpu.async_copy(h_hbm.at[pl.ds(cid * CH, CH)], buf, sem)


def _in_wait(h_hbm, buf, sem):
    pltpu.make_async_copy(h_hbm.at[pl.ds(0, CH)], buf, sem).wait()


def _p1sc_body(h_hbm, w_hbm, out_hbm, wbuf, buf0, buf1, stage, sem0, sem1):
    wid = lax.axis_index("s") * NC + lax.axis_index("c")
    pltpu.sync_copy(w_hbm, wbuf)
    wv = [wbuf[pl.ds(t * L, L)] for t in range(T)]
    start = _worker_start(wid)

    def rows(buf, acc):
        def row_body(r, acc):
            acc = list(acc)
            for p in range(P):
                a = acc[p]
                for t in range(T):
                    a = a + buf[r, p, pl.ds(t * L, L)] * wv[t]
                acc[p] = a
            return tuple(acc)
        return lax.fori_loop(0, CH, row_body, acc)

    _in_start(h_hbm, start, buf0, sem0)

    def body2(k, acc):
        g = 2 * k
        _in_start(h_hbm, start + g + 1, buf1, sem1)
        _in_wait(h_hbm, buf0, sem0)
        acc = rows(buf0, acc)
        _in_start(h_hbm, start + g + 2, buf0, sem0)
        _in_wait(h_hbm, buf1, sem1)
        return rows(buf1, acc)

    acc0 = tuple(jnp.zeros((L,), jnp.float32) for _ in range(P))
    acc = lax.fori_loop(0, HALF, body2, acc0)
    _in_wait(h_hbm, buf0, sem0)
    acc = rows(buf0, acc)
    for p in range(P):
        stage[p, :] = acc[p]

    @pl.when(wid < SC_EXTRA)
    def _extra():
        pltpu.sync_copy(h_hbm.at[pl.ds((start + SC_BASE) * CH, CH)], buf1)
        acc_e = rows(buf1, acc0)
        for p in range(P):
            stage[p, :] = stage[p, :] + acc_e[p]

    pltpu.sync_copy(stage, out_hbm.at[wid])


def _p1sc(h, Wf):
    f = pl.kernel(
        _p1sc_body,
        out_type=jax.ShapeDtypeStruct((NW, P, L), jnp.float32),
        mesh=_sc_mesh(),
        scratch_types=[
            pltpu.VMEM((D,), jnp.float32),
            pltpu.VMEM((CH, P, D), jnp.float32),
            pltpu.VMEM((CH, P, D), jnp.float32),
            pltpu.VMEM((P, L), jnp.float32),
            pltpu.SemaphoreType.DMA,
            pltpu.SemaphoreType.DMA,
        ],
    )
    return f(h, Wf)


def _p1tc_body(h_ref, zphi_ref, acc_ref):
    i = pl.program_id(0)

    @pl.when(i == 0)
    def _init():
        acc_ref[...] = jnp.zeros_like(acc_ref)

    acc_ref[...] += jnp.sum(h_ref[...], axis=0)

    @pl.when(i == NB - 1)
    def _fin():
        zphi_ref[...] = acc_ref[...]


def _p1tc(h):
    return pl.pallas_call(
        _p1tc_body,
        grid=(NB,),
        in_specs=[pl.BlockSpec((BN, P, D), lambda i: (i, 0, 0))],
        out_specs=pl.BlockSpec((P, D), lambda i: (0, 0)),
        out_shape=jax.ShapeDtypeStruct((P, D), jnp.float32),
        scratch_shapes=[pltpu.VMEM((P, D), jnp.float32)],
    )(h)


def _p2tc_body(zphi_ref, w_ref, wpart_ref, h_ref, z_ref, beta_ref):
    i = pl.program_id(0)

    @pl.when(i == 0)
    def _betastep():
        w = jnp.dot(zphi_ref[...], w_ref[...])
        w = w + jnp.sum(wpart_ref[...], axis=(0, 2)).reshape(P, 1)
        w = jnp.where(w >= 0, w, 0.01 * w)
        m = jnp.max(w, axis=0, keepdims=True)
        e = jnp.exp(w - m)
        beta = e / jnp.sum(e, axis=0, keepdims=True)
        beta_ref[...] = jnp.broadcast_to(beta, (P, D))

    z_ref[...] = jnp.sum(h_ref[...] * beta_ref[...][None, :, :], axis=1)


def _p2tc(zphi, Wm, wpart, h):
    return pl.pallas_call(
        _p2tc_body,
        grid=(NB2,),
        in_specs=[
            pl.BlockSpec((P, D), lambda i: (0, 0)),
            pl.BlockSpec((D, 1), lambda i: (0, 0)),
            pl.BlockSpec((NW, P, L), lambda i: (0, 0, 0)),
            pl.BlockSpec((BN2, P, D), lambda i: (i, 0, 0)),
        ],
        out_specs=pl.BlockSpec((BN2, D), lambda i: (i, 0)),
        out_shape=jax.ShapeDtypeStruct((N, D), jnp.float32),
        scratch_shapes=[pltpu.VMEM((P, D), jnp.float32)],
    )(zphi, Wm, wpart, h)


def kernel(h, W):
    wpart = _p1sc(h, W.reshape(D))
    zphi = _p1tc(h)
    return _p2tc(zphi, W, wpart, h)
